# Initial kernel scaffold; baseline (speedup 1.0000x reference)
import jax
import jax.numpy as jnp
from jax.experimental import pallas as pl


def kernel(point_cloud, W1, b1, g1, be1, W2, b2, g2, be2, Wg0, bg0, Wr, br, Wg1, bg1, We, bwe):
    raise NotImplementedError("write your pallas kernel here")


# speedup vs baseline: 19.3884x; 19.3884x over previous
import functools

import jax
import jax.numpy as jnp
from jax import lax
from jax.experimental import pallas as pl
from jax.experimental.pallas import tpu as pltpu
from jax.experimental.pallas import tpu_sc as plsc

_CH = 32
_K = 16


def _leaky(x):
    return jnp.where(x >= 0, x, 0.2 * x)


def _ln(x, g, b):
    m = jnp.mean(x, axis=-1, keepdims=True)
    d = x - m
    v = jnp.mean(d * d, axis=-1, keepdims=True)
    return d * lax.rsqrt(v + 1e-5) * g + b


def _tc_pre_body(p0_ref, q1_ref, W1, b1, g1, be1, W2, b2, g2, be2,
                 Wg0a, Wg0b, bg0, z1_ref, c1_ref):
    x = p0_ref[0]
    h = jnp.dot(x, W1[...], preferred_element_type=jnp.float32) + b1[...]
    h = _leaky(_ln(h, g1[...], be1[...]))
    f = jnp.dot(h, W2[...], preferred_element_type=jnp.float32) + b2[...]
    f = _leaky(_ln(f, g2[...], be2[...]))
    z1_ref[0] = (jnp.dot(x, Wg0a[...], preferred_element_type=jnp.float32)
                 + jnp.dot(f, Wg0b[...], preferred_element_type=jnp.float32))
    q = q1_ref[0]
    c1_ref[0] = jnp.dot(q, Wg0a[...], preferred_element_type=jnp.float32) - bg0[...]


def _tc_mid_body(f1_ref, q1_ref, q2_ref, Wr, br, Wg1a, Wg1b, bg1,
                 z2_ref, c2_ref):
    f1 = f1_ref[0]
    red = _leaky(jnp.dot(f1, Wr[...], preferred_element_type=jnp.float32) + br[...])
    z2_ref[0] = (jnp.dot(q1_ref[0], Wg1a[...], preferred_element_type=jnp.float32)
                 + jnp.dot(red, Wg1b[...], preferred_element_type=jnp.float32))
    c2_ref[0] = jnp.dot(q2_ref[0], Wg1a[...], preferred_element_type=jnp.float32) - bg1[...]


def _tc_post_body(m_ref, We, bwe, o_ref):
    y = _leaky(jnp.dot(m_ref[0], We[...], preferred_element_type=jnp.float32) + bwe[...])
    o_ref[0] = jnp.mean(y, axis=0, keepdims=True)


def _full(shape):
    return pl.BlockSpec(shape, lambda b: (0,) * len(shape))


def _per_b(shape):
    return pl.BlockSpec((1,) + shape, lambda b: (b,) + (0,) * len(shape))


def _make_sc_knn(B, nsrc, nqh):
    mesh = plsc.VectorSubcoreMesh(core_axis_name="c", subcore_axis_name="s")
    n_iter = nsrc // 16

    @functools.partial(
        pl.kernel,
        out_type=jax.ShapeDtypeStruct((B, 2, nqh * _CH), jnp.float32),
        mesh=mesh,
        compiler_params=pltpu.CompilerParams(needs_layout_passes=False),
        scratch_types=[
            pltpu.VMEM((3, nsrc), jnp.float32),
            pltpu.VMEM((3 * nqh,), jnp.float32),
            pltpu.VMEM((nsrc * _CH,), jnp.float32),
            pltpu.VMEM((nqh * _CH,), jnp.float32),
            pltpu.VMEM((nqh * _CH,), jnp.float32),
        ],
    )
    def knn_kernel(src_hbm, q_hbm, z_hbm, c_hbm, out_hbm,
                   src_v, q_v, z_v, c_v, o_v):
        b = lax.axis_index("s")
        hh = lax.axis_index("c")
        pltpu.sync_copy(src_hbm.at[b], src_v)
        pltpu.sync_copy(q_hbm.at[b, hh], q_v)
        pltpu.sync_copy(z_hbm.at[b], z_v)
        pltpu.sync_copy(c_hbm.at[b, hh], c_v)

        iota = lax.iota(jnp.int32, 16)
        inf16 = jnp.full((16,), 3.0e38, jnp.float32)
        zero16 = jnp.zeros((16,), jnp.int32)
        neg16 = jnp.full((16,), -3.0e38, jnp.float32)

        def per_group(g, carry):
            g0 = g * 16
            qxv = q_v[pl.ds(g0, 16)]
            qyv = q_v[pl.ds(nqh + g0, 16)]
            qzv = q_v[pl.ds(2 * nqh + g0, 16)]

            for j in range(16):
                qx, qy, qz = qxv[j], qyv[j], qzv[j]

                def step(t, bdbi):
                    bd, bi = bdbi
                    s0 = t * 16
                    dx = src_v[0, pl.ds(s0, 16)] - qx
                    dy = src_v[1, pl.ds(s0, 16)] - qy
                    dz = src_v[2, pl.ds(s0, 16)] - qz
                    dd = dx * dx + dy * dy + dz * dz
                    cd, ci = plsc.sort_key_val(dd, s0 + iota, descending=True)
                    keep = bd <= cd
                    md = jnp.where(keep, bd, cd)
                    mi = jnp.where(keep, bi, ci)
                    nd, ni = plsc.sort_key_val(md, mi)
                    return nd, ni

                bd, bi = lax.fori_loop(0, n_iter, step, (inf16, zero16))
                base = bi * _CH

                a0, a1 = neg16, neg16
                for j2 in range(_K):
                    ij = base[j2]
                    r0 = z_v[pl.ds(ij, 16)]
                    r1 = z_v[pl.ds(ij + 16, 16)]
                    a0 = jnp.maximum(a0, r0)
                    a1 = jnp.maximum(a1, r1)
                i = g0 + j
                y0 = a0 - c_v[pl.ds(i * _CH, 16)]
                y1 = a1 - c_v[pl.ds(i * _CH + 16, 16)]
                o_v[pl.ds(i * _CH, 16)] = jnp.where(y0 >= 0, y0, 0.2 * y0)
                o_v[pl.ds(i * _CH + 16, 16)] = jnp.where(y1 >= 0, y1, 0.2 * y1)
            return carry

        lax.fori_loop(0, nqh // 16, per_group, 0)
        pltpu.sync_copy(o_v, out_hbm.at[b, hh])

    return knn_kernel


def kernel(point_cloud, W1, b1, g1, be1, W2, b2, g2, be2,
           Wg0, bg0, Wr, br, Wg1, bg1, We, bwe):
    B, _, N = point_cloud.shape
    N1, N2 = N // 2, N // 4
    x0 = point_cloud
    x1 = jnp.asarray(x0[:, :, ::2])
    x2 = jnp.asarray(x0[:, :, ::4])
    p0t = x0.transpose(0, 2, 1)
    q1t = x1.transpose(0, 2, 1)
    q2t = x2.transpose(0, 2, 1)
    q1s = x1.reshape(B, 3, 2, N1 // 2).transpose(0, 2, 1, 3).reshape(B, 2, 3 * (N1 // 2))
    q2s = x2.reshape(B, 3, 2, N2 // 2).transpose(0, 2, 1, 3).reshape(B, 2, 3 * (N2 // 2))

    r1 = lambda a: a.reshape(1, -1)
    b1r, g1r, be1r = r1(b1), r1(g1), r1(be1)
    b2r, g2r, be2r = r1(b2), r1(g2), r1(be2)
    bg0r, br_r, bg1r, bwer = r1(bg0), r1(br), r1(bg1), r1(bwe)
    Wg0a, Wg0b = Wg0[:3], Wg0[3:]
    Wg1a, Wg1b = Wg1[:3], Wg1[3:]

    z1, c1 = pl.pallas_call(
        _tc_pre_body,
        grid=(B,),
        in_specs=[
            _per_b((N, 3)), _per_b((N1, 3)),
            _full(W1.shape), _full(b1r.shape), _full(g1r.shape), _full(be1r.shape),
            _full(W2.shape), _full(b2r.shape), _full(g2r.shape), _full(be2r.shape),
            _full(Wg0a.shape), _full(Wg0b.shape), _full(bg0r.shape),
        ],
        out_specs=[_per_b((N, _CH)), _per_b((N1, _CH))],
        out_shape=[
            jax.ShapeDtypeStruct((B, N, _CH), jnp.float32),
            jax.ShapeDtypeStruct((B, N1, _CH), jnp.float32),
        ],
    )(p0t, q1t, W1, b1r, g1r, be1r, W2, b2r, g2r, be2r, Wg0a, Wg0b, bg0r)

    knn1 = _make_sc_knn(B, N, N1 // 2)
    f1 = knn1(x0, q1s, z1.reshape(B, N * _CH), c1.reshape(B, 2, (N1 // 2) * _CH))
    f1 = f1.reshape(B, N1, _CH)

    z2, c2 = pl.pallas_call(
        _tc_mid_body,
        grid=(B,),
        in_specs=[
            _per_b((N1, _CH)), _per_b((N1, 3)), _per_b((N2, 3)),
            _full(Wr.shape), _full(br_r.shape),
            _full(Wg1a.shape), _full(Wg1b.shape), _full(bg1r.shape),
        ],
        out_specs=[_per_b((N1, _CH)), _per_b((N2, _CH))],
        out_shape=[
            jax.ShapeDtypeStruct((B, N1, _CH), jnp.float32),
            jax.ShapeDtypeStruct((B, N2, _CH), jnp.float32),
        ],
    )(f1, q1t, q2t, Wr, br_r, Wg1a, Wg1b, bg1r)

    knn2 = _make_sc_knn(B, N1, N2 // 2)
    m = knn2(x1, q2s, z2.reshape(B, N1 * _CH), c2.reshape(B, 2, (N2 // 2) * _CH))
    m = m.reshape(B, N2, _CH)

    out = pl.pallas_call(
        _tc_post_body,
        grid=(B,),
        in_specs=[_per_b((N2, _CH)), _full(We.shape), _full(bwer.shape)],
        out_specs=_per_b((1, 256)),
        out_shape=jax.ShapeDtypeStruct((B, 1, 256), jnp.float32),
    )(m, We, bwer)
    return out.reshape(B, 256)
